# Initial kernel scaffold; baseline (speedup 1.0000x reference)
import jax
import jax.numpy as jnp
from jax.experimental import pallas as pl


def kernel(x, ids, num_seg, gamma, beta):
    raise NotImplementedError("write your pallas kernel here")


# speedup vs baseline: 1.4464x; 1.4464x over previous
import functools

import jax
import jax.numpy as jnp
from jax.experimental import pallas as pl
from jax.experimental.pallas import tpu as pltpu

N = 320000
D = 128
S = 10000

STATS_BLOCK = 2000


def _stats_kernel(x_ref, o_ref, acc_ref):
    i = pl.program_id(0)

    @pl.when(i == 0)
    def _():
        acc_ref[...] = jnp.zeros_like(acc_ref)

    xb = x_ref[...]
    acc_ref[0:1, :] += jnp.sum(xb, axis=0, keepdims=True)
    acc_ref[1:2, :] += jnp.sum(xb * xb, axis=0, keepdims=True)

    @pl.when(i == pl.num_programs(0) - 1)
    def _():
        o_ref[...] = acc_ref[...]


def _stats(x):
    return pl.pallas_call(
        _stats_kernel,
        grid=(N // STATS_BLOCK,),
        in_specs=[pl.BlockSpec((STATS_BLOCK, D), lambda i: (i, 0))],
        out_specs=pl.BlockSpec((8, D), lambda i: (0, 0)),
        out_shape=jax.ShapeDtypeStruct((8, D), jnp.float32),
        scratch_shapes=[pltpu.VMEM((8, D), jnp.float32)],
    )(x)


R = 256
CHUNKS = 5
ROWS_BLOCK = R * CHUNKS
W = 256
ACC_ROWS = 10496


def _pool_kernel(stats_ref, gamma_ref, beta_ref, unit_ref,
                 x_ref, ids_ref, o_ref, acc_ref):
    i = pl.program_id(0)

    @pl.when(i == 0)
    def _():
        acc_ref[...] = jnp.zeros_like(acc_ref)

    s = stats_ref[...]
    mean = s[0:1, :] / N
    var = s[1:2, :] / N - mean * mean
    rstd = jax.lax.rsqrt(var + 1e-5)
    scale = gamma_ref[...] * rstd
    shift = beta_ref[...] - mean * scale

    col_iota = jax.lax.broadcasted_iota(jnp.int32, (W, R), 0)

    for c in range(CHUNKS):
        xb = x_ref[pl.ds(c * R, R), :]
        v = xb * scale + shift
        act = jnp.where(v > 0, v, jnp.exp(v) - 1.0).astype(jnp.bfloat16)
        rhs = jnp.concatenate(
            [act, jnp.ones((R, D), dtype=jnp.bfloat16)], axis=1)
        ids = ids_ref[0, 0, pl.ds(c * R, R)].reshape(1, R)

        def cond(carry):
            rem_i, _ = carry
            return jnp.max(rem_i) > 0

        def body(carry):
            rem_i, _ = carry
            remaining = rem_i > 0
            masked = jnp.where(remaining, ids, jnp.int32(1 << 30))
            anchor = (jnp.min(masked) // 8) * 8
            offs = ids - anchor
            sel = remaining & (offs < W)
            offs_masked = jnp.broadcast_to(
                jnp.where(sel, offs, jnp.int32(-1)), (W, R))
            onehot_t = jnp.where(
                col_iota == offs_masked, 1.0, 0.0).astype(jnp.bfloat16)
            contrib = jax.lax.dot_general(
                onehot_t, rhs, (((1,), (0,)), ((), ())),
                preferred_element_type=jnp.float32)
            acc_ref[pl.ds(anchor, W), :] += contrib
            return jnp.where(sel, jnp.int32(0), rem_i), anchor

        jax.lax.while_loop(cond, body, (jnp.ones((1, R), dtype=jnp.int32),
                                        jnp.int32(0)))

    @pl.when(i == pl.num_programs(0) - 1)
    def _():
        sums = acc_ref[0:S, 0:D]
        counts = acc_ref[0:S, D:D + 1]
        o_ref[...] = sums * unit_ref[0, 0] / jnp.maximum(counts, 1.0)


def _pool(stats, gamma, beta, unit, x, ids3):
    return pl.pallas_call(
        _pool_kernel,
        grid=(N // ROWS_BLOCK,),
        in_specs=[
            pl.BlockSpec((8, D), lambda i: (0, 0)),
            pl.BlockSpec((1, D), lambda i: (0, 0)),
            pl.BlockSpec((1, D), lambda i: (0, 0)),
            pl.BlockSpec((1, 1), lambda i: (0, 0)),
            pl.BlockSpec((ROWS_BLOCK, D), lambda i: (i, 0)),
            pl.BlockSpec((1, 1, ROWS_BLOCK), lambda i: (i, 0, 0)),
        ],
        out_specs=pl.BlockSpec((S, D), lambda i: (0, 0)),
        out_shape=jax.ShapeDtypeStruct((S, D), jnp.float32),
        scratch_shapes=[pltpu.VMEM((ACC_ROWS, 2 * D), jnp.float32)],
    )(stats, gamma, beta, unit, x, ids3)


@functools.partial(jax.jit, static_argnames=())
def kernel(x, ids, num_seg, gamma, beta):
    stats = _stats(x)
    ids3 = ids.astype(jnp.int32).reshape(N // ROWS_BLOCK, 1, ROWS_BLOCK)
    unit = (jnp.asarray(num_seg, dtype=jnp.float32) / S).reshape(1, 1)
    return _pool(stats, gamma.reshape(1, D), beta.reshape(1, D),
                 unit, x, ids3)


# speedup vs baseline: 2.9784x; 2.0592x over previous
import functools

import jax
import jax.numpy as jnp
from jax.experimental import pallas as pl
from jax.experimental.pallas import tpu as pltpu

N = 320000
D = 128
S = 10000

STATS_BLOCK = 2000


def _stats_kernel(x_ref, o_ref, acc_ref):
    i = pl.program_id(0)

    @pl.when(i == 0)
    def _():
        acc_ref[...] = jnp.zeros_like(acc_ref)

    xb = x_ref[...]
    acc_ref[0:1, :] += jnp.sum(xb, axis=0, keepdims=True)
    acc_ref[1:2, :] += jnp.sum(xb * xb, axis=0, keepdims=True)

    @pl.when(i == pl.num_programs(0) - 1)
    def _():
        o_ref[...] = acc_ref[...]


def _stats(x):
    return pl.pallas_call(
        _stats_kernel,
        grid=(N // STATS_BLOCK,),
        in_specs=[pl.BlockSpec((STATS_BLOCK, D), lambda i: (i, 0))],
        out_specs=pl.BlockSpec((8, D), lambda i: (0, 0)),
        out_shape=jax.ShapeDtypeStruct((8, D), jnp.float32),
        scratch_shapes=[pltpu.VMEM((8, D), jnp.float32)],
    )(x)


R = 256
CHUNKS = 5
ROWS_BLOCK = R * CHUNKS
W = 256
ACC_ROWS = 10496
NSTEPS = N // ROWS_BLOCK


def _pool_kernel(anchors_ref, stats_ref, gamma_ref, beta_ref, unit_ref,
                 x_ref, ids_ref, o_ref, acc_ref):
    i = pl.program_id(0)

    @pl.when(i == 0)
    def _():
        acc_ref[...] = jnp.zeros_like(acc_ref)

    s = stats_ref[...]
    mean = s[0:1, :] / N
    var = s[1:2, :] / N - mean * mean
    rstd = jax.lax.rsqrt(var + 1e-5)
    scale = gamma_ref[...] * rstd
    shift = beta_ref[...] - mean * scale

    lane_iota = jax.lax.broadcasted_iota(jnp.int32, (R, W), 1)

    def window_pass(rem_i, ids_col, rhs, anchor):
        offs = ids_col - anchor
        sel = (rem_i > 0) & (offs >= 0) & (offs < W)
        offs_m = jnp.broadcast_to(
            jnp.where(sel, offs, jnp.int32(-1)), (R, W))
        onehot = jnp.where(
            lane_iota == offs_m, 1.0, 0.0).astype(jnp.bfloat16)
        contrib = jax.lax.dot_general(
            onehot, rhs, (((0,), (0,)), ((), ())),
            preferred_element_type=jnp.float32)
        acc_ref[pl.ds(anchor, W), :] += contrib
        return jnp.where(sel, jnp.int32(0), rem_i)

    rems = []
    chunk_data = []
    for c in range(CHUNKS):
        xb = x_ref[pl.ds(c * R, R), :]
        v = xb * scale + shift
        act = jnp.where(v > 0, v, jnp.exp(v) - 1.0).astype(jnp.bfloat16)
        rhs = jnp.concatenate(
            [act, jnp.ones((R, D), dtype=jnp.bfloat16)], axis=1)
        ids_col = ids_ref[pl.ds(c * R, R), :]
        anchor = (anchors_ref[i * CHUNKS + c] // 8) * 8
        rem = window_pass(jnp.ones((R, 1), jnp.int32), ids_col, rhs, anchor)
        rems.append(rem)
        chunk_data.append((ids_col, rhs))

    left = rems[0]
    for c in range(1, CHUNKS):
        left = left + rems[c]

    @pl.when(jnp.sum(left) > 0)
    def _():
        for c in range(CHUNKS):
            ids_col, rhs = chunk_data[c]

            def cond(carry):
                rem_i, _ = carry
                return jnp.max(rem_i) > 0

            def body(carry):
                rem_i, _ = carry
                masked = jnp.where(rem_i > 0, ids_col, jnp.int32(1 << 30))
                anchor = (jnp.min(masked) // 8) * 8
                new_rem = window_pass(rem_i, ids_col, rhs, anchor)
                return new_rem, anchor

            jax.lax.while_loop(cond, body, (rems[c], jnp.int32(0)))

    @pl.when(i == pl.num_programs(0) - 1)
    def _():
        sums = acc_ref[0:S, 0:D]
        counts = acc_ref[0:S, D:D + 1]
        o_ref[...] = sums * unit_ref[0, 0] / jnp.maximum(counts, 1.0)


def _pool(anchors, stats, gamma, beta, unit, x, ids_col):
    grid_spec = pltpu.PrefetchScalarGridSpec(
        num_scalar_prefetch=1,
        grid=(NSTEPS,),
        in_specs=[
            pl.BlockSpec((8, D), lambda i, a: (0, 0)),
            pl.BlockSpec((1, D), lambda i, a: (0, 0)),
            pl.BlockSpec((1, D), lambda i, a: (0, 0)),
            pl.BlockSpec((1, 1), lambda i, a: (0, 0)),
            pl.BlockSpec((ROWS_BLOCK, D), lambda i, a: (i, 0)),
            pl.BlockSpec((ROWS_BLOCK, 1), lambda i, a: (i, 0)),
        ],
        out_specs=pl.BlockSpec((S, D), lambda i, a: (0, 0)),
        scratch_shapes=[pltpu.VMEM((ACC_ROWS, 2 * D), jnp.float32)],
    )
    return pl.pallas_call(
        _pool_kernel,
        grid_spec=grid_spec,
        out_shape=jax.ShapeDtypeStruct((S, D), jnp.float32),
    )(anchors, stats, gamma, beta, unit, x, ids_col)


@functools.partial(jax.jit, static_argnames=())
def kernel(x, ids, num_seg, gamma, beta):
    stats = _stats(x)
    ids32 = ids.astype(jnp.int32)
    anchors = (ids32[::R] // 8) * 8
    unit = (jnp.asarray(num_seg, dtype=jnp.float32) / S).reshape(1, 1)
    return _pool(anchors, stats, gamma.reshape(1, D), beta.reshape(1, D),
                 unit, x, ids32.reshape(N, 1))


# speedup vs baseline: 3.0485x; 1.0235x over previous
import functools

import jax
import jax.numpy as jnp
from jax.experimental import pallas as pl
from jax.experimental.pallas import tpu as pltpu

N = 320000
D = 128
S = 10000

STATS_BLOCK = 2000


def _stats_kernel(x_ref, o_ref, acc_ref):
    i = pl.program_id(0)

    @pl.when(i == 0)
    def _():
        acc_ref[...] = jnp.zeros_like(acc_ref)

    xb = x_ref[...]
    acc_ref[0:1, :] += jnp.sum(xb, axis=0, keepdims=True)
    acc_ref[1:2, :] += jnp.sum(xb * xb, axis=0, keepdims=True)

    @pl.when(i == pl.num_programs(0) - 1)
    def _():
        o_ref[...] = acc_ref[...]


def _stats(x):
    return pl.pallas_call(
        _stats_kernel,
        grid=(N // STATS_BLOCK,),
        in_specs=[pl.BlockSpec((STATS_BLOCK, D), lambda i: (i, 0))],
        out_specs=pl.BlockSpec((8, D), lambda i: (0, 0)),
        out_shape=jax.ShapeDtypeStruct((8, D), jnp.float32),
        scratch_shapes=[pltpu.VMEM((8, D), jnp.float32)],
    )(x)


B = 1280
W = 256
ACC_ROWS = 10496
NSTEPS = N // B


def _pool_kernel(anchors_ref, stats_ref, gamma_ref, beta_ref, unit_ref,
                 x_ref, ids_ref, o_ref, acc_ref):
    i = pl.program_id(0)

    @pl.when(i == 0)
    def _():
        acc_ref[...] = jnp.zeros_like(acc_ref)

    s = stats_ref[...]
    mean = s[0:1, :] / N
    var = s[1:2, :] / N - mean * mean
    rstd = jax.lax.rsqrt(var + 1e-5)
    scale = gamma_ref[...] * rstd
    shift = beta_ref[...] - mean * scale

    lane_iota = jax.lax.broadcasted_iota(jnp.int32, (B, W), 1)

    xb = x_ref[...]
    v = xb * scale + shift
    act = jnp.where(v > 0, v, jnp.exp(v) - 1.0).astype(jnp.bfloat16)
    rhs = jnp.concatenate(
        [act, jnp.ones((B, D), dtype=jnp.bfloat16)], axis=1)
    ids_col = ids_ref[...]

    def window_pass(rem_i, anchor):
        offs = ids_col - anchor
        sel = (rem_i > 0) & (offs >= 0) & (offs < W)
        offs_m = jnp.broadcast_to(
            jnp.where(sel, offs, jnp.int32(-1)), (B, W))
        onehot = jnp.where(
            lane_iota == offs_m, 1.0, 0.0).astype(jnp.bfloat16)
        contrib = jax.lax.dot_general(
            onehot, rhs, (((0,), (0,)), ((), ())),
            preferred_element_type=jnp.float32)
        acc_ref[pl.ds(anchor, W), :] += contrib
        return jnp.where(sel, jnp.int32(0), rem_i)

    anchor0 = (anchors_ref[i] // 8) * 8
    rem = window_pass(jnp.ones((B, 1), jnp.int32), anchor0)

    @pl.when(jnp.sum(rem) > 0)
    def _():
        def cond(carry):
            rem_i, _ = carry
            return jnp.max(rem_i) > 0

        def body(carry):
            rem_i, _ = carry
            masked = jnp.where(rem_i > 0, ids_col, jnp.int32(1 << 30))
            anchor = (jnp.min(masked) // 8) * 8
            return window_pass(rem_i, anchor), anchor

        jax.lax.while_loop(cond, body, (rem, jnp.int32(0)))

    @pl.when(i == pl.num_programs(0) - 1)
    def _():
        sums = acc_ref[0:S, 0:D]
        counts = acc_ref[0:S, D:D + 1]
        o_ref[...] = sums * unit_ref[0, 0] / jnp.maximum(counts, 1.0)


def _pool(anchors, stats, gamma, beta, unit, x, ids_col):
    grid_spec = pltpu.PrefetchScalarGridSpec(
        num_scalar_prefetch=1,
        grid=(NSTEPS,),
        in_specs=[
            pl.BlockSpec((8, D), lambda i, a: (0, 0)),
            pl.BlockSpec((1, D), lambda i, a: (0, 0)),
            pl.BlockSpec((1, D), lambda i, a: (0, 0)),
            pl.BlockSpec((1, 1), lambda i, a: (0, 0)),
            pl.BlockSpec((B, D), lambda i, a: (i, 0)),
            pl.BlockSpec((B, 1), lambda i, a: (i, 0)),
        ],
        out_specs=pl.BlockSpec((S, D), lambda i, a: (0, 0)),
        scratch_shapes=[pltpu.VMEM((ACC_ROWS, 2 * D), jnp.float32)],
    )
    return pl.pallas_call(
        _pool_kernel,
        grid_spec=grid_spec,
        out_shape=jax.ShapeDtypeStruct((S, D), jnp.float32),
    )(anchors, stats, gamma, beta, unit, x, ids_col)


@functools.partial(jax.jit, static_argnames=())
def kernel(x, ids, num_seg, gamma, beta):
    stats = _stats(x)
    ids32 = ids.astype(jnp.int32)
    anchors = (ids32[::B] // 8) * 8
    unit = (jnp.asarray(num_seg, dtype=jnp.float32) / S).reshape(1, 1)
    return _pool(anchors, stats, gamma.reshape(1, D), beta.reshape(1, D),
                 unit, x, ids32.reshape(N, 1))


# speedup vs baseline: 3.5847x; 1.1759x over previous
import functools

import jax
import jax.numpy as jnp
from jax.experimental import pallas as pl
from jax.experimental.pallas import tpu as pltpu

N = 320000
D = 128
S = 10000

STATS_BLOCK = 2000


def _stats_kernel(x_ref, o_ref, acc_ref):
    i = pl.program_id(0)

    @pl.when(i == 0)
    def _():
        acc_ref[...] = jnp.zeros_like(acc_ref)

    xb = x_ref[...]
    acc_ref[0:1, :] += jnp.sum(xb, axis=0, keepdims=True)
    acc_ref[1:2, :] += jnp.sum(xb * xb, axis=0, keepdims=True)

    @pl.when(i == pl.num_programs(0) - 1)
    def _():
        o_ref[...] = acc_ref[...]


def _stats(x):
    return pl.pallas_call(
        _stats_kernel,
        grid=(N // STATS_BLOCK,),
        in_specs=[pl.BlockSpec((STATS_BLOCK, D), lambda i: (i, 0))],
        out_specs=pl.BlockSpec((8, D), lambda i: (0, 0)),
        out_shape=jax.ShapeDtypeStruct((8, D), jnp.float32),
        scratch_shapes=[pltpu.VMEM((8, D), jnp.float32)],
    )(x)


B = 2560
W = 256
ACC_ROWS = 10496
NSTEPS = N // B


def _pool_kernel(meta_ref, stats_ref, gamma_ref, beta_ref, unit_ref,
                 x_ref, ids_ref, o_ref, acc_ref):
    i = pl.program_id(0)

    @pl.when(i == 0)
    def _():
        acc_ref[...] = jnp.zeros_like(acc_ref)

    s = stats_ref[...]
    mean = s[0:1, :] / N
    var = s[1:2, :] / N - mean * mean
    rstd = jax.lax.rsqrt(var + 1e-5)
    scale = gamma_ref[...] * rstd
    shift = beta_ref[...] - mean * scale

    lane_iota = jax.lax.broadcasted_iota(jnp.int32, (B, W), 1)

    xb = x_ref[...]
    v = xb * scale + shift
    act = jnp.where(v > 0, v, jnp.exp(v) - 1.0).astype(jnp.bfloat16)
    rhs = jnp.concatenate(
        [act, jnp.ones((B, D), dtype=jnp.bfloat16)], axis=1)
    ids_col = ids_ref[...]

    anchor0 = (meta_ref[2 * i] // 8) * 8
    span = meta_ref[2 * i + 1]

    offs0 = jnp.broadcast_to(ids_col - anchor0, (B, W))
    onehot = jnp.where(
        lane_iota == offs0, 1.0, 0.0).astype(jnp.bfloat16)
    contrib = jax.lax.dot_general(
        onehot, rhs, (((0,), (0,)), ((), ())),
        preferred_element_type=jnp.float32)
    acc_ref[pl.ds(anchor0, W), :] += contrib

    @pl.when(span >= W)
    def _():
        def window_pass(rem_i, anchor):
            offs = ids_col - anchor
            sel = (rem_i > 0) & (offs >= 0) & (offs < W)
            offs_m = jnp.broadcast_to(
                jnp.where(sel, offs, jnp.int32(-1)), (B, W))
            oh = jnp.where(
                lane_iota == offs_m, 1.0, 0.0).astype(jnp.bfloat16)
            c = jax.lax.dot_general(
                oh, rhs, (((0,), (0,)), ((), ())),
                preferred_element_type=jnp.float32)
            acc_ref[pl.ds(anchor, W), :] += c
            return jnp.where(sel, jnp.int32(0), rem_i)

        def cond(carry):
            rem_i, _ = carry
            return jnp.max(rem_i) > 0

        def body(carry):
            rem_i, _ = carry
            masked = jnp.where(rem_i > 0, ids_col, jnp.int32(1 << 30))
            anchor = (jnp.min(masked) // 8) * 8
            return window_pass(rem_i, anchor), anchor

        rem0 = jnp.where(ids_col - anchor0 >= W, 1, 0).astype(jnp.int32)
        jax.lax.while_loop(cond, body, (rem0, jnp.int32(0)))

    @pl.when(i == pl.num_programs(0) - 1)
    def _():
        sums = acc_ref[0:S, 0:D]
        counts = acc_ref[0:S, D:D + 1]
        o_ref[...] = sums * unit_ref[0, 0] / jnp.maximum(counts, 1.0)


def _pool(meta, stats, gamma, beta, unit, x, ids_col):
    grid_spec = pltpu.PrefetchScalarGridSpec(
        num_scalar_prefetch=1,
        grid=(NSTEPS,),
        in_specs=[
            pl.BlockSpec((8, D), lambda i, a: (0, 0)),
            pl.BlockSpec((1, D), lambda i, a: (0, 0)),
            pl.BlockSpec((1, D), lambda i, a: (0, 0)),
            pl.BlockSpec((1, 1), lambda i, a: (0, 0)),
            pl.BlockSpec((B, D), lambda i, a: (i, 0)),
            pl.BlockSpec((B, 1), lambda i, a: (i, 0)),
        ],
        out_specs=pl.BlockSpec((S, D), lambda i, a: (0, 0)),
        scratch_shapes=[pltpu.VMEM((ACC_ROWS, 2 * D), jnp.float32)],
    )
    return pl.pallas_call(
        _pool_kernel,
        grid_spec=grid_spec,
        out_shape=jax.ShapeDtypeStruct((S, D), jnp.float32),
    )(meta, stats, gamma, beta, unit, x, ids_col)


@functools.partial(jax.jit, static_argnames=())
def kernel(x, ids, num_seg, gamma, beta):
    stats = _stats(x)
    ids32 = ids.astype(jnp.int32)
    anchors = (ids32[::B] // 8) * 8
    spans = ids32[B - 1::B] - anchors
    meta = jnp.stack([anchors, spans], axis=1).reshape(-1)
    unit = (jnp.asarray(num_seg, dtype=jnp.float32) / S).reshape(1, 1)
    return _pool(meta, stats, gamma.reshape(1, D), beta.reshape(1, D),
                 unit, x, ids32.reshape(N, 1))


# speedup vs baseline: 3.6428x; 1.0162x over previous
import functools

import jax
import jax.numpy as jnp
from jax.experimental import pallas as pl
from jax.experimental.pallas import tpu as pltpu

N = 320000
D = 128
S = 10000

STATS_BLOCK = 2000


def _stats_kernel(x_ref, o_ref, acc_ref):
    i = pl.program_id(0)

    @pl.when(i == 0)
    def _():
        acc_ref[...] = jnp.zeros_like(acc_ref)

    xb = x_ref[...]
    acc_ref[0:1, :] += jnp.sum(xb, axis=0, keepdims=True)
    acc_ref[1:2, :] += jnp.sum(xb * xb, axis=0, keepdims=True)

    @pl.when(i == pl.num_programs(0) - 1)
    def _():
        o_ref[...] = acc_ref[...]


def _stats(x):
    return pl.pallas_call(
        _stats_kernel,
        grid=(N // STATS_BLOCK,),
        in_specs=[pl.BlockSpec((STATS_BLOCK, D), lambda i: (i, 0))],
        out_specs=pl.BlockSpec((8, D), lambda i: (0, 0)),
        out_shape=jax.ShapeDtypeStruct((8, D), jnp.float32),
        scratch_shapes=[pltpu.VMEM((8, D), jnp.float32)],
    )(x)


B = 2560
W = 128
ACC_ROWS = 10496
NSTEPS = N // B


def _pool_kernel(meta_ref, stats_ref, gamma_ref, beta_ref, unit_ref,
                 x_ref, ids_ref, o_ref, acc_ref):
    i = pl.program_id(0)

    @pl.when(i == 0)
    def _():
        acc_ref[...] = jnp.zeros_like(acc_ref)

    s = stats_ref[...]
    mean = s[0:1, :] / N
    var = s[1:2, :] / N - mean * mean
    rstd = jax.lax.rsqrt(var + 1e-5)
    scale = gamma_ref[...] * rstd
    shift = beta_ref[...] - mean * scale

    lane_iota = jax.lax.broadcasted_iota(jnp.int32, (B, W), 1)

    xb = x_ref[...]
    v = xb * scale + shift
    act = jnp.where(v > 0, v, jnp.exp(v) - 1.0).astype(jnp.bfloat16)
    rhs = jnp.concatenate(
        [act, jnp.ones((B, D), dtype=jnp.bfloat16)], axis=1)
    ids_col = ids_ref[...]

    anchor0 = (meta_ref[2 * i] // 8) * 8
    span = meta_ref[2 * i + 1]

    offs0 = jnp.broadcast_to(ids_col - anchor0, (B, W))
    onehot = jnp.where(
        lane_iota == offs0, 1.0, 0.0).astype(jnp.bfloat16)
    contrib = jax.lax.dot_general(
        onehot, rhs, (((0,), (0,)), ((), ())),
        preferred_element_type=jnp.float32)
    acc_ref[pl.ds(anchor0, W), :] += contrib

    @pl.when(span >= W)
    def _():
        def window_pass(rem_i, anchor):
            offs = ids_col - anchor
            sel = (rem_i > 0) & (offs >= 0) & (offs < W)
            offs_m = jnp.broadcast_to(
                jnp.where(sel, offs, jnp.int32(-1)), (B, W))
            oh = jnp.where(
                lane_iota == offs_m, 1.0, 0.0).astype(jnp.bfloat16)
            c = jax.lax.dot_general(
                oh, rhs, (((0,), (0,)), ((), ())),
                preferred_element_type=jnp.float32)
            acc_ref[pl.ds(anchor, W), :] += c
            return jnp.where(sel, jnp.int32(0), rem_i)

        def cond(carry):
            rem_i, _ = carry
            return jnp.max(rem_i) > 0

        def body(carry):
            rem_i, _ = carry
            masked = jnp.where(rem_i > 0, ids_col, jnp.int32(1 << 30))
            anchor = (jnp.min(masked) // 8) * 8
            return window_pass(rem_i, anchor), anchor

        rem0 = jnp.where(ids_col - anchor0 >= W, 1, 0).astype(jnp.int32)
        jax.lax.while_loop(cond, body, (rem0, jnp.int32(0)))

    @pl.when(i == pl.num_programs(0) - 1)
    def _():
        sums = acc_ref[0:S, 0:D]
        counts = acc_ref[0:S, D:D + 1]
        o_ref[...] = sums * unit_ref[0, 0] / jnp.maximum(counts, 1.0)


def _pool(meta, stats, gamma, beta, unit, x, ids_col):
    grid_spec = pltpu.PrefetchScalarGridSpec(
        num_scalar_prefetch=1,
        grid=(NSTEPS,),
        in_specs=[
            pl.BlockSpec((8, D), lambda i, a: (0, 0)),
            pl.BlockSpec((1, D), lambda i, a: (0, 0)),
            pl.BlockSpec((1, D), lambda i, a: (0, 0)),
            pl.BlockSpec((1, 1), lambda i, a: (0, 0)),
            pl.BlockSpec((B, D), lambda i, a: (i, 0)),
            pl.BlockSpec((B, 1), lambda i, a: (i, 0)),
        ],
        out_specs=pl.BlockSpec((S, D), lambda i, a: (0, 0)),
        scratch_shapes=[pltpu.VMEM((ACC_ROWS, 2 * D), jnp.float32)],
    )
    return pl.pallas_call(
        _pool_kernel,
        grid_spec=grid_spec,
        out_shape=jax.ShapeDtypeStruct((S, D), jnp.float32),
    )(meta, stats, gamma, beta, unit, x, ids_col)


@functools.partial(jax.jit, static_argnames=())
def kernel(x, ids, num_seg, gamma, beta):
    stats = _stats(x)
    ids32 = ids.astype(jnp.int32)
    anchors = (ids32[::B] // 8) * 8
    spans = ids32[B - 1::B] - anchors
    meta = jnp.stack([anchors, spans], axis=1).reshape(-1)
    unit = (jnp.asarray(num_seg, dtype=jnp.float32) / S).reshape(1, 1)
    return _pool(meta, stats, gamma.reshape(1, D), beta.reshape(1, D),
                 unit, x, ids32.reshape(N, 1))


# speedup vs baseline: 4.4453x; 1.2203x over previous
import functools

import jax
import jax.numpy as jnp
from jax.experimental import pallas as pl
from jax.experimental.pallas import tpu as pltpu

N = 320000
D = 128
S = 10000

STATS_BLOCK = 8000


def _stats_kernel(x_ref, o_ref, acc_ref):
    i = pl.program_id(0)

    @pl.when(i == 0)
    def _():
        acc_ref[...] = jnp.zeros_like(acc_ref)

    xb = x_ref[...]
    acc_ref[0:1, :] += jnp.sum(xb, axis=0, keepdims=True)
    acc_ref[1:2, :] += jnp.sum(xb * xb, axis=0, keepdims=True)

    @pl.when(i == pl.num_programs(0) - 1)
    def _():
        o_ref[...] = acc_ref[...]


def _stats(x):
    return pl.pallas_call(
        _stats_kernel,
        grid=(N // STATS_BLOCK,),
        in_specs=[pl.BlockSpec((STATS_BLOCK, D), lambda i: (i, 0))],
        out_specs=pl.BlockSpec((8, D), lambda i: (0, 0)),
        out_shape=jax.ShapeDtypeStruct((8, D), jnp.float32),
        scratch_shapes=[pltpu.VMEM((8, D), jnp.float32)],
    )(x)


B = 4000
W = 256
ACC_ROWS = 10496
NSTEPS = N // B


def _pool_kernel(meta_ref, stats_ref, gamma_ref, beta_ref, unit_ref,
                 x_ref, ids_ref, o_ref, acc_ref):
    i = pl.program_id(0)

    @pl.when(i == 0)
    def _():
        acc_ref[...] = jnp.zeros_like(acc_ref)

    s = stats_ref[...]
    mean = s[0:1, :] / N
    var = s[1:2, :] / N - mean * mean
    rstd = jax.lax.rsqrt(var + 1e-5)
    scale = gamma_ref[...] * rstd
    shift = beta_ref[...] - mean * scale

    lane_iota = jax.lax.broadcasted_iota(jnp.int32, (B, W), 1)

    xb = x_ref[...]
    v = xb * scale + shift
    act = jnp.where(v > 0, v, jnp.exp(v) - 1.0).astype(jnp.bfloat16)
    rhs = jnp.concatenate(
        [act, jnp.ones((B, D), dtype=jnp.bfloat16)], axis=1)
    ids_col = ids_ref[...]

    anchor0 = (meta_ref[2 * i] // 8) * 8
    span = meta_ref[2 * i + 1]

    offs0 = jnp.broadcast_to(ids_col - anchor0, (B, W))
    onehot = jnp.where(
        lane_iota == offs0, 1.0, 0.0).astype(jnp.bfloat16)
    contrib = jax.lax.dot_general(
        onehot, rhs, (((0,), (0,)), ((), ())),
        preferred_element_type=jnp.float32)
    acc_ref[pl.ds(anchor0, W), :] += contrib

    @pl.when(span >= W)
    def _():
        def window_pass(rem_i, anchor):
            offs = ids_col - anchor
            sel = (rem_i > 0) & (offs >= 0) & (offs < W)
            offs_m = jnp.broadcast_to(
                jnp.where(sel, offs, jnp.int32(-1)), (B, W))
            oh = jnp.where(
                lane_iota == offs_m, 1.0, 0.0).astype(jnp.bfloat16)
            c = jax.lax.dot_general(
                oh, rhs, (((0,), (0,)), ((), ())),
                preferred_element_type=jnp.float32)
            acc_ref[pl.ds(anchor, W), :] += c
            return jnp.where(sel, jnp.int32(0), rem_i)

        def cond(carry):
            rem_i, _ = carry
            return jnp.max(rem_i) > 0

        def body(carry):
            rem_i, _ = carry
            masked = jnp.where(rem_i > 0, ids_col, jnp.int32(1 << 30))
            anchor = (jnp.min(masked) // 8) * 8
            return window_pass(rem_i, anchor), anchor

        rem0 = jnp.where(ids_col - anchor0 >= W, 1, 0).astype(jnp.int32)
        jax.lax.while_loop(cond, body, (rem0, jnp.int32(0)))

    @pl.when(i == pl.num_programs(0) - 1)
    def _():
        sums = acc_ref[0:S, 0:D]
        counts = acc_ref[0:S, D:D + 1]
        o_ref[...] = sums * unit_ref[0, 0] / jnp.maximum(counts, 1.0)


def _pool(meta, stats, gamma, beta, unit, x, ids_col):
    grid_spec = pltpu.PrefetchScalarGridSpec(
        num_scalar_prefetch=1,
        grid=(NSTEPS,),
        in_specs=[
            pl.BlockSpec((8, D), lambda i, a: (0, 0)),
            pl.BlockSpec((1, D), lambda i, a: (0, 0)),
            pl.BlockSpec((1, D), lambda i, a: (0, 0)),
            pl.BlockSpec((1, 1), lambda i, a: (0, 0)),
            pl.BlockSpec((B, D), lambda i, a: (i, 0)),
            pl.BlockSpec((B, 1), lambda i, a: (i, 0)),
        ],
        out_specs=pl.BlockSpec((S, D), lambda i, a: (0, 0)),
        scratch_shapes=[pltpu.VMEM((ACC_ROWS, 2 * D), jnp.float32)],
    )
    return pl.pallas_call(
        _pool_kernel,
        grid_spec=grid_spec,
        out_shape=jax.ShapeDtypeStruct((S, D), jnp.float32),
    )(meta, stats, gamma, beta, unit, x, ids_col)


@functools.partial(jax.jit, static_argnames=())
def kernel(x, ids, num_seg, gamma, beta):
    stats = _stats(x)
    ids32 = ids.astype(jnp.int32)
    anchors = (ids32[::B] // 8) * 8
    spans = ids32[B - 1::B] - anchors
    meta = jnp.stack([anchors, spans], axis=1).reshape(-1)
    unit = (jnp.asarray(num_seg, dtype=jnp.float32) / S).reshape(1, 1)
    return _pool(meta, stats, gamma.reshape(1, D), beta.reshape(1, D),
                 unit, x, ids32.reshape(N, 1))


# speedup vs baseline: 4.9395x; 1.1112x over previous
import functools

import jax
import jax.numpy as jnp
from jax.experimental import pallas as pl
from jax.experimental.pallas import tpu as pltpu

N = 320000
D = 128
S = 10000

SB = 8000


def _stats_kernel(xa_ref, xb_ref, o_ref, acc_ref):
    i = pl.program_id(0)

    @pl.when(i == 0)
    def _():
        acc_ref[...] = jnp.zeros_like(acc_ref)

    xa = xa_ref[...]
    xb = xb_ref[...]
    acc_ref[0:1, :] += (jnp.sum(xa, axis=0, keepdims=True)
                        + jnp.sum(xb, axis=0, keepdims=True))
    acc_ref[1:2, :] += (jnp.sum(xa * xa, axis=0, keepdims=True)
                        + jnp.sum(xb * xb, axis=0, keepdims=True))

    @pl.when(i == pl.num_programs(0) - 1)
    def _():
        o_ref[...] = acc_ref[...]


def _stats(x):
    return pl.pallas_call(
        _stats_kernel,
        grid=(N // (2 * SB),),
        in_specs=[
            pl.BlockSpec((SB, D), lambda i: (2 * i, 0)),
            pl.BlockSpec((SB, D), lambda i: (2 * i + 1, 0)),
        ],
        out_specs=pl.BlockSpec((8, D), lambda i: (0, 0)),
        out_shape=jax.ShapeDtypeStruct((8, D), jnp.float32),
        scratch_shapes=[pltpu.VMEM((8, D), jnp.float32)],
    )(x, x)


SUB = 4000
B = 2 * SUB
W = 256
ACC_ROWS = 10496
NSTEPS = N // B


def _pool_kernel(meta_ref, stats_ref, gamma_ref, beta_ref, unit_ref,
                 xa_ref, xb_ref, ida_ref, idb_ref, o_ref, acc_ref):
    i = pl.program_id(0)

    @pl.when(i == 0)
    def _():
        acc_ref[...] = jnp.zeros_like(acc_ref)

    s = stats_ref[...]
    mean = s[0:1, :] / N
    var = s[1:2, :] / N - mean * mean
    rstd = jax.lax.rsqrt(var + 1e-5)
    scale = gamma_ref[...] * rstd
    shift = beta_ref[...] - mean * scale

    lane_iota = jax.lax.broadcasted_iota(jnp.int32, (SUB, W), 1)

    def do_sub(x_ref, ids_ref, k):
        xb = x_ref[...]
        v = xb * scale + shift
        act = jnp.where(v > 0, v, jnp.exp(v) - 1.0).astype(jnp.bfloat16)
        rhs = jnp.concatenate(
            [act, jnp.ones((SUB, D), dtype=jnp.bfloat16)], axis=1)
        ids_col = ids_ref[...]

        anchor0 = (meta_ref[2 * (2 * i + k)] // 8) * 8
        span = meta_ref[2 * (2 * i + k) + 1]

        offs0 = jnp.broadcast_to(ids_col - anchor0, (SUB, W))
        onehot = jnp.where(
            lane_iota == offs0, 1.0, 0.0).astype(jnp.bfloat16)
        contrib = jax.lax.dot_general(
            onehot, rhs, (((0,), (0,)), ((), ())),
            preferred_element_type=jnp.float32)
        acc_ref[pl.ds(anchor0, W), :] += contrib

        @pl.when(span >= W)
        def _():
            def window_pass(rem_i, anchor):
                offs = ids_col - anchor
                sel = (rem_i > 0) & (offs >= 0) & (offs < W)
                offs_m = jnp.broadcast_to(
                    jnp.where(sel, offs, jnp.int32(-1)), (SUB, W))
                oh = jnp.where(
                    lane_iota == offs_m, 1.0, 0.0).astype(jnp.bfloat16)
                c = jax.lax.dot_general(
                    oh, rhs, (((0,), (0,)), ((), ())),
                    preferred_element_type=jnp.float32)
                acc_ref[pl.ds(anchor, W), :] += c
                return jnp.where(sel, jnp.int32(0), rem_i)

            def cond(carry):
                rem_i, _ = carry
                return jnp.max(rem_i) > 0

            def body(carry):
                rem_i, _ = carry
                masked = jnp.where(rem_i > 0, ids_col, jnp.int32(1 << 30))
                anchor = (jnp.min(masked) // 8) * 8
                return window_pass(rem_i, anchor), anchor

            rem0 = jnp.where(ids_col - anchor0 >= W, 1, 0).astype(jnp.int32)
            jax.lax.while_loop(cond, body, (rem0, jnp.int32(0)))

    do_sub(xa_ref, ida_ref, 0)
    do_sub(xb_ref, idb_ref, 1)

    @pl.when(i == pl.num_programs(0) - 1)
    def _():
        sums = acc_ref[0:S, 0:D]
        counts = acc_ref[0:S, D:D + 1]
        o_ref[...] = sums * unit_ref[0, 0] / jnp.maximum(counts, 1.0)


def _pool(meta, stats, gamma, beta, unit, x, ids_col):
    grid_spec = pltpu.PrefetchScalarGridSpec(
        num_scalar_prefetch=1,
        grid=(NSTEPS,),
        in_specs=[
            pl.BlockSpec((8, D), lambda i, a: (0, 0)),
            pl.BlockSpec((1, D), lambda i, a: (0, 0)),
            pl.BlockSpec((1, D), lambda i, a: (0, 0)),
            pl.BlockSpec((1, 1), lambda i, a: (0, 0)),
            pl.BlockSpec((SUB, D), lambda i, a: (2 * i, 0)),
            pl.BlockSpec((SUB, D), lambda i, a: (2 * i + 1, 0)),
            pl.BlockSpec((SUB, 1), lambda i, a: (2 * i, 0)),
            pl.BlockSpec((SUB, 1), lambda i, a: (2 * i + 1, 0)),
        ],
        out_specs=pl.BlockSpec((S, D), lambda i, a: (0, 0)),
        scratch_shapes=[pltpu.VMEM((ACC_ROWS, 2 * D), jnp.float32)],
    )
    return pl.pallas_call(
        _pool_kernel,
        grid_spec=grid_spec,
        out_shape=jax.ShapeDtypeStruct((S, D), jnp.float32),
    )(meta, stats, gamma, beta, unit, x, x, ids_col, ids_col)


@functools.partial(jax.jit, static_argnames=())
def kernel(x, ids, num_seg, gamma, beta):
    stats = _stats(x)
    ids32 = ids.astype(jnp.int32)
    anchors = (ids32[::SUB] // 8) * 8
    spans = ids32[SUB - 1::SUB] - anchors
    meta = jnp.stack([anchors, spans], axis=1).reshape(-1)
    unit = (jnp.asarray(num_seg, dtype=jnp.float32) / S).reshape(1, 1)
    return _pool(meta, stats, gamma.reshape(1, D), beta.reshape(1, D),
                 unit, x, ids32.reshape(N, 1))


# speedup vs baseline: 4.9957x; 1.0114x over previous
import functools

import jax
import jax.numpy as jnp
from jax.experimental import pallas as pl
from jax.experimental.pallas import tpu as pltpu

N = 320000
D = 128
S = 10000

SB = 4000
NSTREAMS = 4


def _stats_kernel(*refs):
    x_refs, o_ref, acc_ref = refs[:NSTREAMS], refs[NSTREAMS], refs[NSTREAMS + 1]
    i = pl.program_id(0)

    @pl.when(i == 0)
    def _():
        acc_ref[...] = jnp.zeros_like(acc_ref)

    for r in x_refs:
        xb = r[...]
        acc_ref[0:1, :] += jnp.sum(xb, axis=0, keepdims=True)
        acc_ref[1:2, :] += jnp.sum(xb * xb, axis=0, keepdims=True)

    @pl.when(i == pl.num_programs(0) - 1)
    def _():
        o_ref[...] = acc_ref[...]


def _make_stats_spec(k):
    return pl.BlockSpec((SB, D), lambda i: (NSTREAMS * i + k, 0))


def _stats(x):
    return pl.pallas_call(
        _stats_kernel,
        grid=(N // (NSTREAMS * SB),),
        in_specs=[_make_stats_spec(k) for k in range(NSTREAMS)],
        out_specs=pl.BlockSpec((8, D), lambda i: (0, 0)),
        out_shape=jax.ShapeDtypeStruct((8, D), jnp.float32),
        scratch_shapes=[pltpu.VMEM((8, D), jnp.float32)],
    )(*([x] * NSTREAMS))


SUB = 4000
NSUB = 4
B = NSUB * SUB
W = 256
ACC_ROWS = 10496
NSTEPS = N // B


def _pool_kernel(meta_ref, stats_ref, gamma_ref, beta_ref, unit_ref,
                 *refs):
    x_refs = refs[:NSUB]
    id_refs = refs[NSUB:2 * NSUB]
    o_ref = refs[2 * NSUB]
    acc_ref = refs[2 * NSUB + 1]
    i = pl.program_id(0)

    @pl.when(i == 0)
    def _():
        acc_ref[...] = jnp.zeros_like(acc_ref)

    s = stats_ref[...]
    mean = s[0:1, :] / N
    var = s[1:2, :] / N - mean * mean
    rstd = jax.lax.rsqrt(var + 1e-5)
    scale = gamma_ref[...] * rstd
    shift = beta_ref[...] - mean * scale

    lane_iota = jax.lax.broadcasted_iota(jnp.int32, (SUB, W), 1)

    def do_sub(x_ref, ids_ref, k):
        xb = x_ref[...]
        v = xb * scale + shift
        act = jnp.where(v > 0, v, jnp.exp(v) - 1.0).astype(jnp.bfloat16)
        rhs = jnp.concatenate(
            [act, jnp.ones((SUB, D), dtype=jnp.bfloat16)], axis=1)
        ids_col = ids_ref[...]

        anchor0 = (meta_ref[2 * (NSUB * i + k)] // 8) * 8
        span = meta_ref[2 * (NSUB * i + k) + 1]

        offs0 = jnp.broadcast_to(ids_col - anchor0, (SUB, W))
        onehot = jnp.where(
            lane_iota == offs0, 1.0, 0.0).astype(jnp.bfloat16)
        contrib = jax.lax.dot_general(
            onehot, rhs, (((0,), (0,)), ((), ())),
            preferred_element_type=jnp.float32)
        acc_ref[pl.ds(anchor0, W), :] += contrib

        @pl.when(span >= W)
        def _():
            def window_pass(rem_i, anchor):
                offs = ids_col - anchor
                sel = (rem_i > 0) & (offs >= 0) & (offs < W)
                offs_m = jnp.broadcast_to(
                    jnp.where(sel, offs, jnp.int32(-1)), (SUB, W))
                oh = jnp.where(
                    lane_iota == offs_m, 1.0, 0.0).astype(jnp.bfloat16)
                c = jax.lax.dot_general(
                    oh, rhs, (((0,), (0,)), ((), ())),
                    preferred_element_type=jnp.float32)
                acc_ref[pl.ds(anchor, W), :] += c
                return jnp.where(sel, jnp.int32(0), rem_i)

            def cond(carry):
                rem_i, _ = carry
                return jnp.max(rem_i) > 0

            def body(carry):
                rem_i, _ = carry
                masked = jnp.where(rem_i > 0, ids_col, jnp.int32(1 << 30))
                anchor = (jnp.min(masked) // 8) * 8
                return window_pass(rem_i, anchor), anchor

            rem0 = jnp.where(ids_col - anchor0 >= W, 1, 0).astype(jnp.int32)
            jax.lax.while_loop(cond, body, (rem0, jnp.int32(0)))

    for k in range(NSUB):
        do_sub(x_refs[k], id_refs[k], k)

    @pl.when(i == pl.num_programs(0) - 1)
    def _():
        sums = acc_ref[0:S, 0:D]
        counts = acc_ref[0:S, D:D + 1]
        o_ref[...] = sums * unit_ref[0, 0] / jnp.maximum(counts, 1.0)


def _make_pool_spec(shape, k):
    return pl.BlockSpec(shape, lambda i, a: (NSUB * i + k, 0))


def _pool(meta, stats, gamma, beta, unit, x, ids_col):
    grid_spec = pltpu.PrefetchScalarGridSpec(
        num_scalar_prefetch=1,
        grid=(NSTEPS,),
        in_specs=[
            pl.BlockSpec((8, D), lambda i, a: (0, 0)),
            pl.BlockSpec((1, D), lambda i, a: (0, 0)),
            pl.BlockSpec((1, D), lambda i, a: (0, 0)),
            pl.BlockSpec((1, 1), lambda i, a: (0, 0)),
        ] + [_make_pool_spec((SUB, D), k) for k in range(NSUB)]
          + [_make_pool_spec((SUB, 1), k) for k in range(NSUB)],
        out_specs=pl.BlockSpec((S, D), lambda i, a: (0, 0)),
        scratch_shapes=[pltpu.VMEM((ACC_ROWS, 2 * D), jnp.float32)],
    )
    return pl.pallas_call(
        _pool_kernel,
        grid_spec=grid_spec,
        out_shape=jax.ShapeDtypeStruct((S, D), jnp.float32),
    )(meta, stats, gamma, beta, unit,
      *([x] * NSUB), *([ids_col] * NSUB))


@functools.partial(jax.jit, static_argnames=())
def kernel(x, ids, num_seg, gamma, beta):
    stats = _stats(x)
    ids32 = ids.astype(jnp.int32)
    anchors = (ids32[::SUB] // 8) * 8
    spans = ids32[SUB - 1::SUB] - anchors
    meta = jnp.stack([anchors, spans], axis=1).reshape(-1)
    unit = (jnp.asarray(num_seg, dtype=jnp.float32) / S).reshape(1, 1)
    return _pool(meta, stats, gamma.reshape(1, D), beta.reshape(1, D),
                 unit, x, ids32.reshape(N, 1))


# speedup vs baseline: 11.2137x; 2.2447x over previous
import functools

import jax
import jax.numpy as jnp
from jax.experimental import pallas as pl
from jax.experimental.pallas import tpu as pltpu

N = 320000
D = 128
S = 10000

SB = 4000
NSTREAMS = 4


def _stats_kernel(*refs):
    x_refs, o_ref, acc_ref = refs[:NSTREAMS], refs[NSTREAMS], refs[NSTREAMS + 1]
    i = pl.program_id(0)

    @pl.when(i == 0)
    def _():
        acc_ref[...] = jnp.zeros_like(acc_ref)

    for r in x_refs:
        xb = r[...]
        acc_ref[0:1, :] += jnp.sum(xb, axis=0, keepdims=True)
        acc_ref[1:2, :] += jnp.sum(xb * xb, axis=0, keepdims=True)

    @pl.when(i == pl.num_programs(0) - 1)
    def _():
        o_ref[...] = acc_ref[...]


def _make_stats_spec(k):
    return pl.BlockSpec((SB, D), lambda i: (NSTREAMS * i + k, 0))


def _stats(x):
    return pl.pallas_call(
        _stats_kernel,
        grid=(N // (NSTREAMS * SB),),
        in_specs=[_make_stats_spec(k) for k in range(NSTREAMS)],
        out_specs=pl.BlockSpec((8, D), lambda i: (0, 0)),
        out_shape=jax.ShapeDtypeStruct((8, D), jnp.float32),
        scratch_shapes=[pltpu.VMEM((8, D), jnp.float32)],
    )(*([x] * NSTREAMS))


SUB = 3200
NSUB = 4
B = NSUB * SUB
W = 256
ACC_ROWS = 10496
NSTEPS = N // B
NB = N // SUB


def _pool_kernel(meta_ref, stats_ref, gamma_ref, beta_ref, unit_ref,
                 *refs):
    x_refs = refs[:NSUB]
    id_refs = refs[NSUB:2 * NSUB]
    o_ref = refs[2 * NSUB]
    acc_ref = refs[2 * NSUB + 1]
    i = pl.program_id(0)

    @pl.when(i == 0)
    def _():
        acc_ref[...] = jnp.zeros_like(acc_ref)

    s = stats_ref[...]
    mean = s[0:1, :] / N
    var = s[1:2, :] / N - mean * mean
    rstd = jax.lax.rsqrt(var + 1e-5)
    scale = gamma_ref[...] * rstd
    shift = beta_ref[...] - mean * scale

    sub_iota = jax.lax.broadcasted_iota(jnp.int32, (W, SUB), 0)

    def do_sub(x_ref, ids_ref, k):
        xb = x_ref[...]
        v = xb * scale + shift
        act = jnp.where(v > 0, v, jnp.exp(v) - 1.0).astype(jnp.bfloat16)
        rhs = jnp.concatenate(
            [act, jnp.ones((SUB, D), dtype=jnp.bfloat16)], axis=1)
        ids_row = ids_ref[0]

        anchor0 = (meta_ref[2 * (NSUB * i + k)] // 8) * 8
        span = meta_ref[2 * (NSUB * i + k) + 1]

        offs0 = jnp.broadcast_to(ids_row - anchor0, (W, SUB))
        onehot = jnp.where(
            sub_iota == offs0, 1.0, 0.0).astype(jnp.bfloat16)
        contrib = jax.lax.dot_general(
            onehot, rhs, (((1,), (0,)), ((), ())),
            preferred_element_type=jnp.float32)
        acc_ref[pl.ds(anchor0, W), :] += contrib

        @pl.when(span >= W)
        def _():
            def window_pass(rem_i, anchor):
                offs = ids_row - anchor
                sel = (rem_i > 0) & (offs >= 0) & (offs < W)
                offs_m = jnp.broadcast_to(
                    jnp.where(sel, offs, jnp.int32(-1)), (W, SUB))
                oh = jnp.where(
                    sub_iota == offs_m, 1.0, 0.0).astype(jnp.bfloat16)
                c = jax.lax.dot_general(
                    oh, rhs, (((1,), (0,)), ((), ())),
                    preferred_element_type=jnp.float32)
                acc_ref[pl.ds(anchor, W), :] += c
                return jnp.where(sel, jnp.int32(0), rem_i)

            def cond(carry):
                rem_i, _ = carry
                return jnp.max(rem_i) > 0

            def body(carry):
                rem_i, _ = carry
                masked = jnp.where(rem_i > 0, ids_row, jnp.int32(1 << 30))
                anchor = (jnp.min(masked) // 8) * 8
                return window_pass(rem_i, anchor), anchor

            rem0 = jnp.where(ids_row - anchor0 >= W, 1, 0).astype(jnp.int32)
            jax.lax.while_loop(cond, body, (rem0, jnp.int32(0)))

    for k in range(NSUB):
        do_sub(x_refs[k], id_refs[k], k)

    @pl.when(i == pl.num_programs(0) - 1)
    def _():
        sums = acc_ref[0:S, 0:D]
        counts = acc_ref[0:S, D:D + 1]
        o_ref[...] = sums * unit_ref[0, 0] / jnp.maximum(counts, 1.0)


def _make_pool_x_spec(k):
    return pl.BlockSpec((SUB, D), lambda i, a: (NSUB * i + k, 0))


def _make_pool_id_spec(k):
    return pl.BlockSpec((1, 1, SUB), lambda i, a: (NSUB * i + k, 0, 0))


def _pool(meta, stats, gamma, beta, unit, x, ids3):
    grid_spec = pltpu.PrefetchScalarGridSpec(
        num_scalar_prefetch=1,
        grid=(NSTEPS,),
        in_specs=[
            pl.BlockSpec((8, D), lambda i, a: (0, 0)),
            pl.BlockSpec((1, D), lambda i, a: (0, 0)),
            pl.BlockSpec((1, D), lambda i, a: (0, 0)),
            pl.BlockSpec((1, 1), lambda i, a: (0, 0)),
        ] + [_make_pool_x_spec(k) for k in range(NSUB)]
          + [_make_pool_id_spec(k) for k in range(NSUB)],
        out_specs=pl.BlockSpec((S, D), lambda i, a: (0, 0)),
        scratch_shapes=[pltpu.VMEM((ACC_ROWS, 2 * D), jnp.float32)],
    )
    return pl.pallas_call(
        _pool_kernel,
        grid_spec=grid_spec,
        out_shape=jax.ShapeDtypeStruct((S, D), jnp.float32),
    )(meta, stats, gamma, beta, unit,
      *([x] * NSUB), *([ids3] * NSUB))


@functools.partial(jax.jit, static_argnames=())
def kernel(x, ids, num_seg, gamma, beta):
    stats = _stats(x)
    ids32 = ids.astype(jnp.int32)
    anchors = (ids32[::SUB] // 8) * 8
    spans = ids32[SUB - 1::SUB] - anchors
    meta = jnp.stack([anchors, spans], axis=1).reshape(-1)
    unit = (jnp.asarray(num_seg, dtype=jnp.float32) / S).reshape(1, 1)
    ids3 = ids32.reshape(NB, 1, SUB)
    return _pool(meta, stats, gamma.reshape(1, D), beta.reshape(1, D),
                 unit, x, ids3)
